# Initial kernel scaffold; baseline (speedup 1.0000x reference)
import jax
import jax.numpy as jnp
from jax.experimental import pallas as pl


def kernel(x, var, embeddings):
    raise NotImplementedError("write your pallas kernel here")


# speedup vs baseline: 1.2493x; 1.2493x over previous
import functools

import jax
import jax.numpy as jnp
from jax import lax
from jax.experimental import pallas as pl
from jax.experimental.pallas import tpu as pltpu
from jax.experimental.pallas import tpu_sc as plsc

N_TOKENS = 16384
NUM_EMBEDDINGS = 8192
EMBEDDING_DIM = 64
COMMITMENT_COST = 0.25

TOKEN_TILE = 1024
CODE_TILE = 1024


def _argmin_body(xx_ref, ee_ref, x_ref, e_ref, idx_ref, loss_ref):
    i = pl.program_id(0)
    xb = x_ref[...].astype(jnp.bfloat16)
    xx = xx_ref[...]
    n_chunks = NUM_EMBEDDINGS // CODE_TILE

    iota = lax.broadcasted_iota(jnp.int32, (TOKEN_TILE, CODE_TILE), 1)
    big = jnp.int32(2**30)

    halves = []
    for h in range(2):
        best = jnp.full((TOKEN_TILE, 1), jnp.inf, dtype=jnp.float32)
        bidx = jnp.zeros((TOKEN_TILE, 1), dtype=jnp.int32)
        for jj in range(n_chunks // 2):
            j = h * (n_chunks // 2) + jj
            ej = e_ref[j * CODE_TILE:(j + 1) * CODE_TILE, :].astype(
                jnp.bfloat16)
            eej = ee_ref[j:j + 1, :]
            m = lax.dot_general(xb, ej, (((1,), (1,)), ((), ())),
                                preferred_element_type=jnp.float32)
            d = (xx + eej) - 2.0 * m
            cmin = jnp.min(d, axis=1, keepdims=True)
            cidx = jnp.min(jnp.where(d == cmin, iota, big), axis=1,
                           keepdims=True) + jnp.int32(j * CODE_TILE)
            upd = cmin < best
            best = jnp.where(upd, cmin, best)
            bidx = jnp.where(upd, cidx, bidx)
        halves.append((best, bidx))

    (best_a, idx_a), (best_b, idx_b) = halves
    take_b = best_b < best_a.astype(jnp.bfloat16).astype(jnp.float32)
    bidx = jnp.where(take_b, idx_b, idx_a)
    best = jnp.where(take_b, best_b, best_a)

    idx_ref[...] = bidx
    @pl.when(i == 0)
    def _():
        loss_ref[...] = jnp.zeros((1, 1), jnp.float32)
    loss_ref[...] += jnp.sum(best, keepdims=True)


def _distance_argmin(xx, ee, x, embeddings):
    n_tiles = N_TOKENS // TOKEN_TILE
    idx, loss_sum = pl.pallas_call(
        _argmin_body,
        grid=(n_tiles,),
        in_specs=[
            pl.BlockSpec((TOKEN_TILE, 1), lambda i: (i, 0)),
            pl.BlockSpec((NUM_EMBEDDINGS // CODE_TILE, CODE_TILE),
                         lambda i: (0, 0)),
            pl.BlockSpec((TOKEN_TILE, EMBEDDING_DIM), lambda i: (i, 0)),
            pl.BlockSpec((NUM_EMBEDDINGS, EMBEDDING_DIM), lambda i: (0, 0)),
        ],
        out_specs=[
            pl.BlockSpec((TOKEN_TILE, 1), lambda i: (i, 0)),
            pl.BlockSpec((1, 1), lambda i: (0, 0)),
        ],
        out_shape=[
            jax.ShapeDtypeStruct((N_TOKENS, 1), jnp.int32),
            jax.ShapeDtypeStruct((1, 1), jnp.float32),
        ],
    )(xx, ee, x, embeddings)
    return idx, loss_sum


def _sc_gather(table, idx):
    width = 128
    info = plsc.get_sparse_core_info()
    nw = info.num_cores * info.num_subcores
    b_per_w = N_TOKENS // nw
    mesh = plsc.VectorSubcoreMesh(core_axis_name="c", subcore_axis_name="s")

    @functools.partial(
        pl.kernel,
        mesh=mesh,
        out_type=jax.ShapeDtypeStruct((N_TOKENS, width), jnp.float32),
        scratch_types=[
            pltpu.VMEM((b_per_w,), jnp.int32),
            pltpu.VMEM((b_per_w, width), jnp.float32),
            pltpu.SemaphoreType.DMA,
        ],
    )
    def k(table_hbm, idx_hbm, out_hbm, idx_v, rows_v, sem):
        wid = lax.axis_index("s") * info.num_cores + lax.axis_index("c")
        base = wid * b_per_w
        pltpu.sync_copy(idx_hbm.at[pl.ds(base, b_per_w)], idx_v)
        pltpu.async_copy(table_hbm.at[idx_v], rows_v, sem).wait()
        pltpu.sync_copy(rows_v, out_hbm.at[pl.ds(base, b_per_w)])

    padded = jnp.pad(table, ((0, 0), (0, width - EMBEDDING_DIM)))
    return k(padded, idx)[:, :EMBEDDING_DIM]


def kernel(x, var, embeddings):
    del var
    xx = jnp.sum(x ** 2, axis=1, keepdims=True)
    ee = jnp.sum(embeddings ** 2, axis=1).reshape(
        NUM_EMBEDDINGS // CODE_TILE, CODE_TILE)
    idx2d, loss_sum = _distance_argmin(xx, ee, x, embeddings)
    idx = idx2d.reshape(N_TOKENS)
    quantized = _sc_gather(embeddings, idx)
    loss = loss_sum[0, 0] * (1.25 / (N_TOKENS * EMBEDDING_DIM))
    return quantized, loss, idx


# speedup vs baseline: 1.3620x; 1.0902x over previous
import functools

import jax
import jax.numpy as jnp
from jax import lax
from jax.experimental import pallas as pl
from jax.experimental.pallas import tpu as pltpu
from jax.experimental.pallas import tpu_sc as plsc

N_TOKENS = 16384
NUM_EMBEDDINGS = 8192
EMBEDDING_DIM = 64
COMMITMENT_COST = 0.25

TOKEN_TILE = 1024
CODE_TILE = 1024


def _argmin_body(xx_ref, ee_ref, x_ref, e_ref, idx_ref, loss_ref):
    i = pl.program_id(0)
    xb = x_ref[...].astype(jnp.bfloat16)
    xxr = xx_ref[...].reshape(1, TOKEN_TILE)
    n_chunks = NUM_EMBEDDINGS // CODE_TILE

    iota = lax.broadcasted_iota(jnp.int32, (CODE_TILE, TOKEN_TILE), 0)
    big = jnp.int32(2**30)

    halves = []
    for h in range(2):
        best = jnp.full((1, TOKEN_TILE), jnp.inf, dtype=jnp.float32)
        bidx = jnp.zeros((1, TOKEN_TILE), dtype=jnp.int32)
        for jj in range(n_chunks // 2):
            j = h * (n_chunks // 2) + jj
            ej = e_ref[j * CODE_TILE:(j + 1) * CODE_TILE, :].astype(
                jnp.bfloat16)
            eej = ee_ref[j * CODE_TILE:(j + 1) * CODE_TILE, :]
            m = lax.dot_general(ej, xb, (((1,), (1,)), ((), ())),
                                preferred_element_type=jnp.float32)
            d = (eej + xxr) - 2.0 * m
            cmin = jnp.min(d, axis=0, keepdims=True)
            cidx = jnp.min(jnp.where(d == cmin, iota, big), axis=0,
                           keepdims=True) + jnp.int32(j * CODE_TILE)
            upd = cmin < best
            best = jnp.where(upd, cmin, best)
            bidx = jnp.where(upd, cidx, bidx)
        halves.append((best, bidx))

    (best_a, idx_a), (best_b, idx_b) = halves
    take_b = best_b < best_a.astype(jnp.bfloat16).astype(jnp.float32)
    bidx = jnp.where(take_b, idx_b, idx_a)
    best = jnp.where(take_b, best_b, best_a)

    idx_ref[...] = bidx.reshape(1, 1, TOKEN_TILE)
    @pl.when(i == 0)
    def _():
        loss_ref[...] = jnp.zeros((1, 1), jnp.float32)
    loss_ref[...] += jnp.sum(best, keepdims=True)


def _distance_argmin(xx, ee, x, embeddings):
    n_tiles = N_TOKENS // TOKEN_TILE
    idx, loss_sum = pl.pallas_call(
        _argmin_body,
        grid=(n_tiles,),
        in_specs=[
            pl.BlockSpec((1, 1, TOKEN_TILE), lambda i: (i, 0, 0)),
            pl.BlockSpec((NUM_EMBEDDINGS, 1), lambda i: (0, 0)),
            pl.BlockSpec((TOKEN_TILE, EMBEDDING_DIM), lambda i: (i, 0)),
            pl.BlockSpec((NUM_EMBEDDINGS, EMBEDDING_DIM), lambda i: (0, 0)),
        ],
        out_specs=[
            pl.BlockSpec((1, 1, TOKEN_TILE), lambda i: (i, 0, 0)),
            pl.BlockSpec((1, 1), lambda i: (0, 0)),
        ],
        out_shape=[
            jax.ShapeDtypeStruct((n_tiles, 1, TOKEN_TILE), jnp.int32),
            jax.ShapeDtypeStruct((1, 1), jnp.float32),
        ],
    )(xx, ee, x, embeddings)
    return idx, loss_sum


def _sc_gather(table, idx):
    width = 128
    info = plsc.get_sparse_core_info()
    nw = info.num_cores * info.num_subcores
    b_per_w = N_TOKENS // nw
    mesh = plsc.VectorSubcoreMesh(core_axis_name="c", subcore_axis_name="s")

    @functools.partial(
        pl.kernel,
        mesh=mesh,
        out_type=jax.ShapeDtypeStruct((N_TOKENS, width), jnp.float32),
        scratch_types=[
            pltpu.VMEM((b_per_w,), jnp.int32),
            pltpu.VMEM((b_per_w, width), jnp.float32),
            pltpu.SemaphoreType.DMA,
        ],
    )
    def k(table_hbm, idx_hbm, out_hbm, idx_v, rows_v, sem):
        wid = lax.axis_index("s") * info.num_cores + lax.axis_index("c")
        base = wid * b_per_w
        pltpu.sync_copy(idx_hbm.at[pl.ds(base, b_per_w)], idx_v)
        pltpu.async_copy(table_hbm.at[idx_v], rows_v, sem).wait()
        pltpu.sync_copy(rows_v, out_hbm.at[pl.ds(base, b_per_w)])

    padded = jnp.pad(table, ((0, 0), (0, width - EMBEDDING_DIM)))
    return k(padded, idx)[:, :EMBEDDING_DIM]


def kernel(x, var, embeddings):
    del var
    xx = jnp.sum(x ** 2, axis=1).reshape(
        N_TOKENS // TOKEN_TILE, 1, TOKEN_TILE)
    ee = jnp.sum(embeddings ** 2, axis=1).reshape(
        NUM_EMBEDDINGS, 1)
    idx2d, loss_sum = _distance_argmin(xx, ee, x, embeddings)
    idx = idx2d.reshape(N_TOKENS)
    quantized = _sc_gather(embeddings, idx)
    loss = loss_sum[0, 0] * (1.25 / (N_TOKENS * EMBEDDING_DIM))
    return quantized, loss, idx


# speedup vs baseline: 2.0443x; 1.5010x over previous
import functools

import jax
import jax.numpy as jnp
from jax import lax
from jax.experimental import pallas as pl
from jax.experimental.pallas import tpu as pltpu
from jax.experimental.pallas import tpu_sc as plsc

N_TOKENS = 16384
NUM_EMBEDDINGS = 8192
EMBEDDING_DIM = 64
COMMITMENT_COST = 0.25

TOKEN_TILE = 1024
CODE_TILE = 1024


def _argmin_body(xx_ref, ee_ref, xb_ref, em2_ref, idx_ref, loss_ref):
    i = pl.program_id(0)
    xb = xb_ref[...]
    xxr = xx_ref[...].reshape(1, TOKEN_TILE)
    n_chunks = NUM_EMBEDDINGS // CODE_TILE
    rg = CODE_TILE // 8

    sub_iota = lax.broadcasted_iota(jnp.int32, (8, TOKEN_TILE), 0)

    halves = []
    for h in range(2):
        acc_v = jnp.full((8, TOKEN_TILE), jnp.inf, dtype=jnp.float32)
        acc_r = jnp.zeros((8, TOKEN_TILE), dtype=jnp.int32)
        for jj in range(n_chunks // 2):
            j = h * (n_chunks // 2) + jj
            ej = em2_ref[j * CODE_TILE:(j + 1) * CODE_TILE, :]
            eej = ee_ref[j * CODE_TILE:(j + 1) * CODE_TILE, :]
            m2 = lax.dot_general(ej, xb, (((1,), (1,)), ((), ())),
                                 preferred_element_type=jnp.float32)
            for r in range(rg):
                sl = slice(r * 8, (r + 1) * 8)
                dg = (eej[sl, :] + xxr) + m2[sl, :]
                take = dg < acc_v
                acc_r = jnp.where(take, jnp.int32(j * rg + r), acc_r)
                acc_v = jnp.where(take, dg, acc_v)
        code = acc_r * 8 + sub_iota
        v, c = acc_v, code
        k = 8
        while k > 1:
            k //= 2
            v_lo, v_hi = v[:k, :], v[k:, :]
            c_lo, c_hi = c[:k, :], c[k:, :]
            t = (v_hi < v_lo) | ((v_hi == v_lo) & (c_hi < c_lo))
            v = jnp.where(t, v_hi, v_lo)
            c = jnp.where(t, c_hi, c_lo)
        halves.append((v, c))

    (best_a, idx_a), (best_b, idx_b) = halves
    take_b = best_b < best_a.astype(jnp.bfloat16).astype(jnp.float32)
    bidx = jnp.where(take_b, idx_b, idx_a)
    best = jnp.where(take_b, best_b, best_a)

    idx_ref[...] = bidx.reshape(1, 1, TOKEN_TILE)
    @pl.when(i == 0)
    def _():
        loss_ref[...] = jnp.zeros((1, 1), jnp.float32)
    loss_ref[...] += jnp.sum(best, keepdims=True)


def _distance_argmin(xx, ee, x, embeddings):
    n_tiles = N_TOKENS // TOKEN_TILE
    idx, loss_sum = pl.pallas_call(
        _argmin_body,
        grid=(n_tiles,),
        in_specs=[
            pl.BlockSpec((1, 1, TOKEN_TILE), lambda i: (i, 0, 0)),
            pl.BlockSpec((NUM_EMBEDDINGS, 1), lambda i: (0, 0)),
            pl.BlockSpec((TOKEN_TILE, EMBEDDING_DIM), lambda i: (i, 0)),
            pl.BlockSpec((NUM_EMBEDDINGS, EMBEDDING_DIM), lambda i: (0, 0)),
        ],
        name="vq_argmin",
        out_specs=[
            pl.BlockSpec((1, 1, TOKEN_TILE), lambda i: (i, 0, 0)),
            pl.BlockSpec((1, 1), lambda i: (0, 0)),
        ],
        out_shape=[
            jax.ShapeDtypeStruct((n_tiles, 1, TOKEN_TILE), jnp.int32),
            jax.ShapeDtypeStruct((1, 1), jnp.float32),
        ],
    )(xx, ee, x, embeddings)
    return idx, loss_sum


def _sc_gather(table, idx):
    width = 128
    info = plsc.get_sparse_core_info()
    nw = info.num_cores * info.num_subcores
    b_per_w = N_TOKENS // nw
    mesh = plsc.VectorSubcoreMesh(core_axis_name="c", subcore_axis_name="s")

    @functools.partial(
        pl.kernel,
        mesh=mesh,
        out_type=jax.ShapeDtypeStruct((N_TOKENS, width), jnp.float32),
        scratch_types=[
            pltpu.VMEM((b_per_w,), jnp.int32),
            pltpu.VMEM((b_per_w, width), jnp.float32),
            pltpu.SemaphoreType.DMA,
        ],
    )
    def k(table_hbm, idx_hbm, out_hbm, idx_v, rows_v, sem):
        wid = lax.axis_index("s") * info.num_cores + lax.axis_index("c")
        base = wid * b_per_w
        pltpu.sync_copy(idx_hbm.at[pl.ds(base, b_per_w)], idx_v)
        pltpu.async_copy(table_hbm.at[idx_v], rows_v, sem).wait()
        pltpu.sync_copy(rows_v, out_hbm.at[pl.ds(base, b_per_w)])

    padded = jnp.pad(table, ((0, 0), (0, width - EMBEDDING_DIM)))
    return k(padded, idx)[:, :EMBEDDING_DIM]


def kernel(x, var, embeddings):
    del var
    xx = jnp.sum(x ** 2, axis=1).reshape(
        N_TOKENS // TOKEN_TILE, 1, TOKEN_TILE)
    ee = jnp.sum(embeddings ** 2, axis=1).reshape(
        NUM_EMBEDDINGS, 1)
    xb = x.astype(jnp.bfloat16)
    em2 = (-2.0 * embeddings).astype(jnp.bfloat16)
    idx2d, loss_sum = _distance_argmin(xx, ee, xb, em2)
    idx = idx2d.reshape(N_TOKENS)
    quantized = _sc_gather(embeddings, idx)
    loss = loss_sum[0, 0] * (1.25 / (N_TOKENS * EMBEDDING_DIM))
    return quantized, loss, idx


# speedup vs baseline: 2.0993x; 1.0269x over previous
import functools

import jax
import jax.numpy as jnp
from jax import lax
from jax.experimental import pallas as pl
from jax.experimental.pallas import tpu as pltpu
from jax.experimental.pallas import tpu_sc as plsc

N_TOKENS = 16384
NUM_EMBEDDINGS = 8192
EMBEDDING_DIM = 64
COMMITMENT_COST = 0.25

TOKEN_TILE = 2048
CODE_TILE = 1024


def _argmin_body(xx_ref, ee_ref, xb_ref, em2_ref, idx_ref, loss_ref):
    i = pl.program_id(0)
    xb = xb_ref[...]
    xxr = xx_ref[...].reshape(1, TOKEN_TILE)
    n_chunks = NUM_EMBEDDINGS // CODE_TILE
    rg = CODE_TILE // 8

    sub_iota = lax.broadcasted_iota(jnp.int32, (8, TOKEN_TILE), 0)

    halves = []
    for h in range(2):
        acc_v = jnp.full((8, TOKEN_TILE), jnp.inf, dtype=jnp.float32)
        acc_r = jnp.zeros((8, TOKEN_TILE), dtype=jnp.int32)
        for jj in range(n_chunks // 2):
            j = h * (n_chunks // 2) + jj
            ej = em2_ref[j * CODE_TILE:(j + 1) * CODE_TILE, :]
            eej = ee_ref[j * CODE_TILE:(j + 1) * CODE_TILE, :]
            m2 = lax.dot_general(ej, xb, (((1,), (1,)), ((), ())),
                                 preferred_element_type=jnp.float32)
            for r in range(rg):
                sl = slice(r * 8, (r + 1) * 8)
                dg = (eej[sl, :] + xxr) + m2[sl, :]
                take = dg < acc_v
                acc_r = jnp.where(take, jnp.int32(j * rg + r), acc_r)
                acc_v = jnp.where(take, dg, acc_v)
        code = acc_r * 8 + sub_iota
        v, c = acc_v, code
        k = 8
        while k > 1:
            k //= 2
            v_lo, v_hi = v[:k, :], v[k:, :]
            c_lo, c_hi = c[:k, :], c[k:, :]
            t = (v_hi < v_lo) | ((v_hi == v_lo) & (c_hi < c_lo))
            v = jnp.where(t, v_hi, v_lo)
            c = jnp.where(t, c_hi, c_lo)
        halves.append((v, c))

    (best_a, idx_a), (best_b, idx_b) = halves
    take_b = best_b < best_a.astype(jnp.bfloat16).astype(jnp.float32)
    bidx = jnp.where(take_b, idx_b, idx_a)
    best = jnp.where(take_b, best_b, best_a)

    idx_ref[...] = bidx.reshape(1, 1, TOKEN_TILE)
    @pl.when(i == 0)
    def _():
        loss_ref[...] = jnp.zeros((1, 1), jnp.float32)
    loss_ref[...] += jnp.sum(best, keepdims=True)


def _distance_argmin(xx, ee, x, embeddings):
    n_tiles = N_TOKENS // TOKEN_TILE
    idx, loss_sum = pl.pallas_call(
        _argmin_body,
        grid=(n_tiles,),
        in_specs=[
            pl.BlockSpec((1, 1, TOKEN_TILE), lambda i: (i, 0, 0)),
            pl.BlockSpec((NUM_EMBEDDINGS, 1), lambda i: (0, 0)),
            pl.BlockSpec((TOKEN_TILE, EMBEDDING_DIM), lambda i: (i, 0)),
            pl.BlockSpec((NUM_EMBEDDINGS, EMBEDDING_DIM), lambda i: (0, 0)),
        ],
        name="vq_argmin",
        out_specs=[
            pl.BlockSpec((1, 1, TOKEN_TILE), lambda i: (i, 0, 0)),
            pl.BlockSpec((1, 1), lambda i: (0, 0)),
        ],
        out_shape=[
            jax.ShapeDtypeStruct((n_tiles, 1, TOKEN_TILE), jnp.int32),
            jax.ShapeDtypeStruct((1, 1), jnp.float32),
        ],
    )(xx, ee, x, embeddings)
    return idx, loss_sum


def _sc_gather(table, idx):
    width = 128
    info = plsc.get_sparse_core_info()
    nw = info.num_cores * info.num_subcores
    b_per_w = N_TOKENS // nw
    mesh = plsc.VectorSubcoreMesh(core_axis_name="c", subcore_axis_name="s")

    @functools.partial(
        pl.kernel,
        mesh=mesh,
        out_type=jax.ShapeDtypeStruct((N_TOKENS, width), jnp.float32),
        scratch_types=[
            pltpu.VMEM((b_per_w,), jnp.int32),
            pltpu.VMEM((b_per_w, width), jnp.float32),
            pltpu.SemaphoreType.DMA,
        ],
    )
    def k(table_hbm, idx_hbm, out_hbm, idx_v, rows_v, sem):
        wid = lax.axis_index("s") * info.num_cores + lax.axis_index("c")
        base = wid * b_per_w
        pltpu.sync_copy(idx_hbm.at[pl.ds(base, b_per_w)], idx_v)
        pltpu.async_copy(table_hbm.at[idx_v], rows_v, sem).wait()
        pltpu.sync_copy(rows_v, out_hbm.at[pl.ds(base, b_per_w)])

    padded = jnp.pad(table, ((0, 0), (0, width - EMBEDDING_DIM)))
    return k(padded, idx)[:, :EMBEDDING_DIM]


def kernel(x, var, embeddings):
    del var
    xx = jnp.sum(x ** 2, axis=1).reshape(
        N_TOKENS // TOKEN_TILE, 1, TOKEN_TILE)
    ee = jnp.sum(embeddings ** 2, axis=1).reshape(
        NUM_EMBEDDINGS, 1)
    xb = x.astype(jnp.bfloat16)
    em2 = (-2.0 * embeddings).astype(jnp.bfloat16)
    idx2d, loss_sum = _distance_argmin(xx, ee, xb, em2)
    idx = idx2d.reshape(N_TOKENS)
    quantized = _sc_gather(embeddings, idx)
    loss = loss_sum[0, 0] * (1.25 / (N_TOKENS * EMBEDDING_DIM))
    return quantized, loss, idx


# speedup vs baseline: 2.1365x; 1.0177x over previous
import functools

import jax
import jax.numpy as jnp
from jax import lax
from jax.experimental import pallas as pl
from jax.experimental.pallas import tpu as pltpu
from jax.experimental.pallas import tpu_sc as plsc

N_TOKENS = 16384
NUM_EMBEDDINGS = 8192
EMBEDDING_DIM = 64
COMMITMENT_COST = 0.25

TOKEN_TILE = 4096
CODE_TILE = 1024


def _argmin_body(xx_ref, ee_ref, xb_ref, em2_ref, idx_ref, loss_ref):
    i = pl.program_id(0)
    xb = xb_ref[...]
    xxr = xx_ref[...].reshape(1, TOKEN_TILE)
    n_chunks = NUM_EMBEDDINGS // CODE_TILE
    rg = CODE_TILE // 8

    sub_iota = lax.broadcasted_iota(jnp.int32, (8, TOKEN_TILE), 0)

    halves = []
    for h in range(2):
        acc_v = jnp.full((8, TOKEN_TILE), jnp.inf, dtype=jnp.float32)
        acc_r = jnp.zeros((8, TOKEN_TILE), dtype=jnp.int32)
        for jj in range(n_chunks // 2):
            j = h * (n_chunks // 2) + jj
            ej = em2_ref[j * CODE_TILE:(j + 1) * CODE_TILE, :]
            eej = ee_ref[j * CODE_TILE:(j + 1) * CODE_TILE, :]
            m2 = lax.dot_general(ej, xb, (((1,), (1,)), ((), ())),
                                 preferred_element_type=jnp.float32)
            for r in range(rg):
                sl = slice(r * 8, (r + 1) * 8)
                dg = (eej[sl, :] + xxr) + m2[sl, :]
                take = dg < acc_v
                acc_r = jnp.where(take, jnp.int32(j * rg + r), acc_r)
                acc_v = jnp.where(take, dg, acc_v)
        code = acc_r * 8 + sub_iota
        v, c = acc_v, code
        k = 8
        while k > 1:
            k //= 2
            v_lo, v_hi = v[:k, :], v[k:, :]
            c_lo, c_hi = c[:k, :], c[k:, :]
            t = (v_hi < v_lo) | ((v_hi == v_lo) & (c_hi < c_lo))
            v = jnp.where(t, v_hi, v_lo)
            c = jnp.where(t, c_hi, c_lo)
        halves.append((v, c))

    (best_a, idx_a), (best_b, idx_b) = halves
    take_b = best_b < best_a.astype(jnp.bfloat16).astype(jnp.float32)
    bidx = jnp.where(take_b, idx_b, idx_a)
    best = jnp.where(take_b, best_b, best_a)

    idx_ref[...] = bidx.reshape(1, 1, TOKEN_TILE)
    @pl.when(i == 0)
    def _():
        loss_ref[...] = jnp.zeros((1, 1), jnp.float32)
    loss_ref[...] += jnp.sum(best, keepdims=True)


def _distance_argmin(xx, ee, x, embeddings):
    n_tiles = N_TOKENS // TOKEN_TILE
    idx, loss_sum = pl.pallas_call(
        _argmin_body,
        grid=(n_tiles,),
        in_specs=[
            pl.BlockSpec((1, 1, TOKEN_TILE), lambda i: (i, 0, 0)),
            pl.BlockSpec((NUM_EMBEDDINGS, 1), lambda i: (0, 0)),
            pl.BlockSpec((TOKEN_TILE, EMBEDDING_DIM), lambda i: (i, 0)),
            pl.BlockSpec((NUM_EMBEDDINGS, EMBEDDING_DIM), lambda i: (0, 0)),
        ],
        name="vq_argmin",
        out_specs=[
            pl.BlockSpec((1, 1, TOKEN_TILE), lambda i: (i, 0, 0)),
            pl.BlockSpec((1, 1), lambda i: (0, 0)),
        ],
        out_shape=[
            jax.ShapeDtypeStruct((n_tiles, 1, TOKEN_TILE), jnp.int32),
            jax.ShapeDtypeStruct((1, 1), jnp.float32),
        ],
    )(xx, ee, x, embeddings)
    return idx, loss_sum


def _sc_gather(table, idx):
    width = 128
    info = plsc.get_sparse_core_info()
    nw = info.num_cores * info.num_subcores
    b_per_w = N_TOKENS // nw
    mesh = plsc.VectorSubcoreMesh(core_axis_name="c", subcore_axis_name="s")

    @functools.partial(
        pl.kernel,
        mesh=mesh,
        out_type=jax.ShapeDtypeStruct((N_TOKENS, width), jnp.float32),
        scratch_types=[
            pltpu.VMEM((b_per_w,), jnp.int32),
            pltpu.VMEM((b_per_w, width), jnp.float32),
            pltpu.SemaphoreType.DMA,
        ],
    )
    def k(table_hbm, idx_hbm, out_hbm, idx_v, rows_v, sem):
        wid = lax.axis_index("s") * info.num_cores + lax.axis_index("c")
        base = wid * b_per_w
        pltpu.sync_copy(idx_hbm.at[pl.ds(base, b_per_w)], idx_v)
        pltpu.async_copy(table_hbm.at[idx_v], rows_v, sem).wait()
        pltpu.sync_copy(rows_v, out_hbm.at[pl.ds(base, b_per_w)])

    padded = jnp.pad(table, ((0, 0), (0, width - EMBEDDING_DIM)))
    return k(padded, idx)[:, :EMBEDDING_DIM]


def kernel(x, var, embeddings):
    del var
    xx = jnp.sum(x ** 2, axis=1).reshape(
        N_TOKENS // TOKEN_TILE, 1, TOKEN_TILE)
    ee = jnp.sum(embeddings ** 2, axis=1).reshape(
        NUM_EMBEDDINGS, 1)
    xb = x.astype(jnp.bfloat16)
    em2 = (-2.0 * embeddings).astype(jnp.bfloat16)
    idx2d, loss_sum = _distance_argmin(xx, ee, xb, em2)
    idx = idx2d.reshape(N_TOKENS)
    quantized = _sc_gather(embeddings, idx)
    loss = loss_sum[0, 0] * (1.25 / (N_TOKENS * EMBEDDING_DIM))
    return quantized, loss, idx
